# baseline (device time: 22513 ns/iter reference)
import jax
import jax.numpy as jnp
from jax import lax
from jax.experimental import pallas as pl
from jax.experimental.pallas import tpu as pltpu

C = 8


def kernel(x):
    m, n = x.shape
    half = m // 2
    chunk = half // C

    def body(x_ref, out_ref, send1, recv1, s1, r1, s2, r2):
        my_x = lax.axis_index("x")
        my_y = lax.axis_index("y")
        xp = (1 - my_x, my_y)
        yp = (my_x, 1 - my_y)

        barrier_sem = pltpu.get_barrier_semaphore()
        for nbr in (xp, yp):
            pl.semaphore_signal(
                barrier_sem, inc=1, device_id=nbr,
                device_id_type=pl.DeviceIdType.MESH,
            )
        pl.semaphore_wait(barrier_sem, 2)

        rdma1 = []
        for c in range(C):
            send1[c] = x_ref[pl.ds(my_y * half + c * chunk, chunk), :].astype(
                jnp.bfloat16
            )
            d = pltpu.make_async_remote_copy(
                src_ref=send1.at[c],
                dst_ref=recv1.at[c],
                send_sem=s1.at[c],
                recv_sem=r1.at[c],
                device_id=xp,
                device_id_type=pl.DeviceIdType.MESH,
            )
            d.start()
            rdma1.append(d)

        rdma2 = []
        for c in range(C):
            rdma1[c].wait_recv()
            rows = pl.ds(my_y * half + c * chunk, chunk)
            out_ref[rows, :] = send1[c] + recv1[c]
            d = pltpu.make_async_remote_copy(
                src_ref=out_ref.at[rows],
                dst_ref=out_ref.at[rows],
                send_sem=s2.at[c],
                recv_sem=r2.at[c],
                device_id=yp,
                device_id_type=pl.DeviceIdType.MESH,
            )
            d.start()
            rdma2.append(d)

        for c in range(C):
            other = pl.ds((1 - my_y) * half + c * chunk, chunk)
            recv = pltpu.make_async_remote_copy(
                src_ref=out_ref.at[other],
                dst_ref=out_ref.at[other],
                send_sem=s2.at[c],
                recv_sem=r2.at[c],
                device_id=yp,
                device_id_type=pl.DeviceIdType.MESH,
            )
            recv.wait_recv()

        for c in range(C):
            rdma1[c].wait_send()
            rdma2[c].wait_send()

    return pl.pallas_call(
        body,
        out_shape=jax.ShapeDtypeStruct((m, n), jnp.bfloat16),
        in_specs=[pl.BlockSpec(memory_space=pltpu.VMEM)],
        out_specs=pl.BlockSpec(memory_space=pltpu.VMEM),
        scratch_shapes=[
            pltpu.VMEM((C, chunk, n), jnp.bfloat16),
            pltpu.VMEM((C, chunk, n), jnp.bfloat16),
            pltpu.SemaphoreType.DMA((C,)),
            pltpu.SemaphoreType.DMA((C,)),
            pltpu.SemaphoreType.DMA((C,)),
            pltpu.SemaphoreType.DMA((C,)),
        ],
        compiler_params=pltpu.CompilerParams(collective_id=0),
    )(x)
